# baseline (speedup 1.0000x reference)
import functools
import math

import jax
import jax.numpy as jnp
from jax.experimental import pallas as pl
from jax.experimental.pallas import tpu as pltpu
from jax.experimental.pallas import tpu_sc as plsc

K = 9
G = K - 1
QBLK = 1024


def _knn_kernel(q_ref, k_ref, idx_ref, *, n):
    q = q_ref[0]
    keys = k_ref[0]
    qx, qy, qz = q[0:1, :], q[1:2, :], q[2:3, :]
    kx, ky, kz = keys[:, 0:1], keys[:, 1:2], keys[:, 2:3]

    sqq = qx * qx + qy * qy + qz * qz
    sqk = kx * kx + ky * ky + kz * kz
    dot = jax.lax.dot_general(
        keys.astype(jnp.bfloat16), q.astype(jnp.bfloat16),
        (((1,), (0,)), ((), ())),
        preferred_element_type=jnp.float32)
    dist = sqq + sqk - 2.0 * dot

    row_iota = jax.lax.broadcasted_iota(jnp.int32, (n, dist.shape[1]), 0)
    big = jnp.float32(1e30)
    rows = []
    for t in range(K):
        m = jnp.min(dist, axis=0, keepdims=True)
        cand = jnp.where(dist == m, row_iota, n)
        first = jnp.min(cand, axis=0, keepdims=True)
        if t > 0:
            rows.append(first)
        if t < K - 1:
            dist = jnp.where(cand == first, big, dist)

    base = pl.program_id(0) * n
    idx_ref[0] = jnp.concatenate(rows, axis=0) + base


_SORT_NET = [(0, 1), (2, 3), (4, 5), (6, 7),
             (0, 2), (1, 3), (4, 6), (5, 7),
             (1, 2), (5, 6),
             (0, 4), (1, 5), (2, 6), (3, 7),
             (2, 4), (3, 5),
             (1, 2), (3, 4), (5, 6)]


def _umbrella_kernel(gx_ref, gy_ref, gz_ref, c_ref,
                     w1_ref, g1_ref, b1_ref,
                     w2_ref, cb2_ref, g2_ref, b2_ref,
                     w3_ref, cb3_ref, out_ref, *, b, n):
    cx = c_ref[:, 0, :]
    cy = c_ref[:, 1, :]
    cz = c_ref[:, 2, :]

    inv2pi = 1.0 / (2.0 * math.pi)
    invpi = 1.0 / math.pi

    rx = [gx_ref[:, k, :] - cx for k in range(G)]
    ry = [gy_ref[:, k, :] - cy for k in range(G)]
    rz = [gz_ref[:, k, :] - cz for k in range(G)]
    phi = [jnp.arctan2(ry[k], rx[k]) * inv2pi + 0.5 for k in range(G)]
    kidx = [jnp.full((b, n), k, dtype=jnp.int32) for k in range(G)]

    for i, j in _SORT_NET:
        swap = (phi[i] > phi[j]) | ((phi[i] == phi[j]) & (kidx[i] > kidx[j]))
        for arr in (phi, rx, ry, rz, kidx):
            ai, aj = arr[i], arr[j]
            arr[i] = jnp.where(swap, aj, ai)
            arr[j] = jnp.where(swap, ai, aj)

    nx, ny, nz = [], [], []
    ccx, ccy, ccz = [], [], []
    for k in range(G):
        kn = (k + 1) % G
        v1x, v1y, v1z = rx[k], ry[k], rz[k]
        v2x, v2y, v2z = rx[kn], ry[kn], rz[kn]
        cxp = v1y * v2z - v1z * v2y
        cyp = v1z * v2x - v1x * v2z
        czp = v1x * v2y - v1y * v2x
        norm = jnp.sqrt(cxp * cxp + cyp * cyp + czp * czp)
        nx.append(cxp / norm)
        ny.append(cyp / norm)
        nz.append(czp / norm)
        third = jnp.float32(1.0 / 3.0)
        ccx.append((v1x + v2x) * third)
        ccy.append((v1y + v2y) * third)
        ccz.append((v1z + v2z) * third)

    pm = jnp.where(nx[0] > 0.0, 1.0, -1.0).astype(jnp.float32)
    nx = [v * pm for v in nx]
    ny = [v * pm for v in ny]
    nz = [v * pm for v in nz]

    rho, theta, phic = [], [], []
    for k in range(G):
        r2 = ccx[k] * ccx[k] + ccy[k] * ccy[k] + ccz[k] * ccz[k]
        r = jnp.sqrt(r2)
        rs = jnp.where(r == 0.0, 1.0, r)
        ratio = jnp.clip(ccz[k] / rs, -1.0, 1.0)
        acos = jnp.arctan2(jnp.sqrt(jnp.maximum(1.0 - ratio * ratio, 0.0)),
                           ratio)
        th = jnp.where(r == 0.0, 0.0, acos) * invpi
        ph = jnp.arctan2(ccy[k], ccx[k]) * inv2pi + 0.5
        rho.append(r)
        theta.append(th)
        phic.append(ph)

    nanmask = [(nx[k] != nx[k]) | (ny[k] != ny[k]) | (nz[k] != nz[k])
               for k in range(G)]
    mf = jnp.full((b, n), G, dtype=jnp.int32)
    for k in range(G - 1, -1, -1):
        mf = jnp.where(~nanmask[k], jnp.int32(k), mf)
    mf = jnp.where(mf == G, 0, mf)

    def _pick(rows):
        acc = rows[G - 1]
        for k in range(G - 2, -1, -1):
            acc = jnp.where(mf == k, rows[k], acc)
        return acc

    fx, fy, fz = _pick(nx), _pick(ny), _pick(nz)
    fcx, fcy, fcz = _pick(ccx), _pick(ccy), _pick(ccz)
    for k in range(G):
        msk = nanmask[k]
        nx[k] = jnp.where(msk, fx, nx[k])
        ny[k] = jnp.where(msk, fy, ny[k])
        nz[k] = jnp.where(msk, fz, nz[k])
        ccx[k] = jnp.where(msk, fcx, ccx[k])
        ccy[k] = jnp.where(msk, fcy, ccy[k])
        ccz[k] = jnp.where(msk, fcz, ccz[k])

    feat = [ccx, ccy, ccz, rho, theta, phic, nx, ny, nz]

    nsamp = jnp.float32(b * G * n)

    def _conv(xrows, w_ref, cb_ref):
        out = []
        for o in range(9):
            orows = []
            for k in range(G):
                acc = None
                for c in range(9):
                    term = w_ref[o, c] * xrows[c][k]
                    acc = term if acc is None else acc + term
                if cb_ref is not None:
                    acc = acc + cb_ref[o]
                orows.append(acc)
            out.append(orows)
        return out

    def _bn_relu(yrows, g_ref, bb_ref):
        out = []
        for o in range(9):
            s = None
            for k in range(G):
                t = jnp.sum(yrows[o][k])
                s = t if s is None else s + t
            m = s / nsamp
            v = None
            for k in range(G):
                d = yrows[o][k] - m
                t = jnp.sum(d * d)
                v = t if v is None else v + t
            v = v / nsamp
            scale = g_ref[o] / jnp.sqrt(v + 1e-5)
            off = bb_ref[o]
            out.append([jnp.maximum((yrows[o][k] - m) * scale + off, 0.0)
                        for k in range(G)])
        return out

    x = _bn_relu(_conv(feat, w1_ref, None), g1_ref, b1_ref)
    x = _bn_relu(_conv(x, w2_ref, cb2_ref), g2_ref, b2_ref)
    y = _conv(x, w3_ref, cb3_ref)

    for o in range(9):
        acc = y[o][0]
        for k in range(1, G):
            acc = acc + y[o][k]
        out_ref[:, o, :] = acc


_TABW = 16


def _make_sc_gather(total):
    info = plsc.get_sparse_core_info()
    nw = info.num_cores * info.num_subcores
    per_w = total // nw
    mesh = plsc.VectorSubcoreMesh(core_axis_name="c", subcore_axis_name="s")

    @functools.partial(
        pl.kernel, mesh=mesh,
        out_type=jax.ShapeDtypeStruct((total, _TABW), jnp.float32),
        scratch_types=[
            pltpu.VMEM((per_w,), jnp.int32),
            pltpu.VMEM((per_w, _TABW), jnp.float32),
            pltpu.SemaphoreType.DMA,
        ],
        compiler_params=pltpu.CompilerParams(use_tc_tiling_on_sc=False),
    )
    def sc_gather(tab_hbm, idx_hbm, out_hbm, idx_v, rows_v, sem):
        wid = jax.lax.axis_index("s") * info.num_cores + jax.lax.axis_index("c")
        base = wid * per_w
        pltpu.sync_copy(idx_hbm.at[pl.ds(base, per_w)], idx_v)
        pltpu.async_copy(tab_hbm.at[idx_v], rows_v, sem).wait()
        pltpu.sync_copy(rows_v, out_hbm.at[pl.ds(base, per_w)])

    return sc_gather


def kernel(center, conv1_w, bn1_g, bn1_b, conv2_w, conv2_b, bn2_g, bn2_b,
           conv3_w, conv3_b):
    b, _, n = center.shape
    xyz = jnp.transpose(center, (0, 2, 1))

    knn = pl.pallas_call(
        functools.partial(_knn_kernel, n=n),
        grid=(b, n // QBLK),
        in_specs=[
            pl.BlockSpec((1, 3, QBLK), lambda bi, qi: (bi, 0, qi)),
            pl.BlockSpec((1, n, 3), lambda bi, qi: (bi, 0, 0)),
        ],
        out_specs=pl.BlockSpec((1, G, QBLK), lambda bi, qi: (bi, 0, qi)),
        out_shape=jax.ShapeDtypeStruct((b, G, n), jnp.int32),
    )
    idx = knn(center, xyz)

    total = b * G * n
    tab = jnp.zeros((b * n, _TABW), jnp.float32)
    tab = tab.at[:, :3].set(xyz.reshape(b * n, 3))
    rows = _make_sc_gather(total)(tab, idx.reshape(-1))
    gx = rows[:, 0].reshape(b, G, n)
    gy = rows[:, 1].reshape(b, G, n)
    gz = rows[:, 2].reshape(b, G, n)

    smem = pl.BlockSpec(memory_space=pltpu.SMEM)
    vmem = pl.BlockSpec(memory_space=pltpu.VMEM)
    out = pl.pallas_call(
        functools.partial(_umbrella_kernel, b=b, n=n),
        in_specs=[vmem, vmem, vmem, vmem,
                  smem, smem, smem, smem, smem, smem, smem, smem, smem],
        out_specs=vmem,
        out_shape=jax.ShapeDtypeStruct((b, 9, n), jnp.float32),
    )(gx, gy, gz, center,
      conv1_w, bn1_g, bn1_b,
      conv2_w, conv2_b, bn2_g, bn2_b,
      conv3_w, conv3_b)
    return out


# speedup vs baseline: 1.3779x; 1.3779x over previous
import functools
import math

import jax
import jax.numpy as jnp
from jax.experimental import pallas as pl
from jax.experimental.pallas import tpu as pltpu
from jax.experimental.pallas import tpu_sc as plsc

K = 9
G = K - 1
QBLK = 512


def _knn_kernel(q_ref, k_ref, idx_ref, *, n):
    q = q_ref[0]
    keys = k_ref[0]
    qx, qy, qz = q[0:1, :], q[1:2, :], q[2:3, :]
    kx, ky, kz = keys[:, 0:1], keys[:, 1:2], keys[:, 2:3]

    sqq = qx * qx + qy * qy + qz * qz
    sqk = kx * kx + ky * ky + kz * kz
    dot = jax.lax.dot_general(
        keys.astype(jnp.bfloat16), q.astype(jnp.bfloat16),
        (((1,), (0,)), ((), ())),
        preferred_element_type=jnp.float32)
    dist = sqq + sqk - 2.0 * dot

    row_iota = jax.lax.broadcasted_iota(jnp.int32, (n, dist.shape[1]), 0)
    big = jnp.float32(1e30)
    rows = []
    for t in range(K):
        first = jnp.argmin(dist, axis=0, keepdims=True).astype(jnp.int32)
        if t > 0:
            rows.append(first)
        if t < K - 1:
            dist = jnp.where(row_iota == first, big, dist)

    base = pl.program_id(0) * n
    idx_ref[0] = jnp.concatenate(rows, axis=0) + base


_SORT_NET = [(0, 1), (2, 3), (4, 5), (6, 7),
             (0, 2), (1, 3), (4, 6), (5, 7),
             (1, 2), (5, 6),
             (0, 4), (1, 5), (2, 6), (3, 7),
             (2, 4), (3, 5),
             (1, 2), (3, 4), (5, 6)]


def _umbrella_kernel(gx_ref, gy_ref, gz_ref, c_ref,
                     w1_ref, g1_ref, b1_ref,
                     w2_ref, cb2_ref, g2_ref, b2_ref,
                     w3_ref, cb3_ref, out_ref, *, b, n):
    cx = c_ref[:, 0, :]
    cy = c_ref[:, 1, :]
    cz = c_ref[:, 2, :]

    inv2pi = 1.0 / (2.0 * math.pi)
    invpi = 1.0 / math.pi

    rx = [gx_ref[:, k, :] - cx for k in range(G)]
    ry = [gy_ref[:, k, :] - cy for k in range(G)]
    rz = [gz_ref[:, k, :] - cz for k in range(G)]
    phi = [jnp.arctan2(ry[k], rx[k]) * inv2pi + 0.5 for k in range(G)]
    kidx = [jnp.full((b, n), k, dtype=jnp.int32) for k in range(G)]

    for i, j in _SORT_NET:
        swap = (phi[i] > phi[j]) | ((phi[i] == phi[j]) & (kidx[i] > kidx[j]))
        for arr in (phi, rx, ry, rz, kidx):
            ai, aj = arr[i], arr[j]
            arr[i] = jnp.where(swap, aj, ai)
            arr[j] = jnp.where(swap, ai, aj)

    nx, ny, nz = [], [], []
    ccx, ccy, ccz = [], [], []
    for k in range(G):
        kn = (k + 1) % G
        v1x, v1y, v1z = rx[k], ry[k], rz[k]
        v2x, v2y, v2z = rx[kn], ry[kn], rz[kn]
        cxp = v1y * v2z - v1z * v2y
        cyp = v1z * v2x - v1x * v2z
        czp = v1x * v2y - v1y * v2x
        norm = jnp.sqrt(cxp * cxp + cyp * cyp + czp * czp)
        nx.append(cxp / norm)
        ny.append(cyp / norm)
        nz.append(czp / norm)
        third = jnp.float32(1.0 / 3.0)
        ccx.append((v1x + v2x) * third)
        ccy.append((v1y + v2y) * third)
        ccz.append((v1z + v2z) * third)

    pm = jnp.where(nx[0] > 0.0, 1.0, -1.0).astype(jnp.float32)
    nx = [v * pm for v in nx]
    ny = [v * pm for v in ny]
    nz = [v * pm for v in nz]

    rho, theta, phic = [], [], []
    for k in range(G):
        r2 = ccx[k] * ccx[k] + ccy[k] * ccy[k] + ccz[k] * ccz[k]
        r = jnp.sqrt(r2)
        rs = jnp.where(r == 0.0, 1.0, r)
        ratio = jnp.clip(ccz[k] / rs, -1.0, 1.0)
        acos = jnp.arctan2(jnp.sqrt(jnp.maximum(1.0 - ratio * ratio, 0.0)),
                           ratio)
        th = jnp.where(r == 0.0, 0.0, acos) * invpi
        ph = jnp.arctan2(ccy[k], ccx[k]) * inv2pi + 0.5
        rho.append(r)
        theta.append(th)
        phic.append(ph)

    nanmask = [(nx[k] != nx[k]) | (ny[k] != ny[k]) | (nz[k] != nz[k])
               for k in range(G)]
    mf = jnp.full((b, n), G, dtype=jnp.int32)
    for k in range(G - 1, -1, -1):
        mf = jnp.where(~nanmask[k], jnp.int32(k), mf)
    mf = jnp.where(mf == G, 0, mf)

    def _pick(rows):
        acc = rows[G - 1]
        for k in range(G - 2, -1, -1):
            acc = jnp.where(mf == k, rows[k], acc)
        return acc

    fx, fy, fz = _pick(nx), _pick(ny), _pick(nz)
    fcx, fcy, fcz = _pick(ccx), _pick(ccy), _pick(ccz)
    for k in range(G):
        msk = nanmask[k]
        nx[k] = jnp.where(msk, fx, nx[k])
        ny[k] = jnp.where(msk, fy, ny[k])
        nz[k] = jnp.where(msk, fz, nz[k])
        ccx[k] = jnp.where(msk, fcx, ccx[k])
        ccy[k] = jnp.where(msk, fcy, ccy[k])
        ccz[k] = jnp.where(msk, fcz, ccz[k])

    feat = [ccx, ccy, ccz, rho, theta, phic, nx, ny, nz]

    nsamp = jnp.float32(b * G * n)

    def _conv(xrows, w_ref, cb_ref):
        out = []
        for o in range(9):
            orows = []
            for k in range(G):
                acc = None
                for c in range(9):
                    term = w_ref[o, c] * xrows[c][k]
                    acc = term if acc is None else acc + term
                if cb_ref is not None:
                    acc = acc + cb_ref[o]
                orows.append(acc)
            out.append(orows)
        return out

    def _bn_relu(yrows, g_ref, bb_ref):
        out = []
        for o in range(9):
            s = None
            for k in range(G):
                t = jnp.sum(yrows[o][k])
                s = t if s is None else s + t
            m = s / nsamp
            v = None
            for k in range(G):
                d = yrows[o][k] - m
                t = jnp.sum(d * d)
                v = t if v is None else v + t
            v = v / nsamp
            scale = g_ref[o] / jnp.sqrt(v + 1e-5)
            off = bb_ref[o]
            out.append([jnp.maximum((yrows[o][k] - m) * scale + off, 0.0)
                        for k in range(G)])
        return out

    x = _bn_relu(_conv(feat, w1_ref, None), g1_ref, b1_ref)
    x = _bn_relu(_conv(x, w2_ref, cb2_ref), g2_ref, b2_ref)
    y = _conv(x, w3_ref, cb3_ref)

    for o in range(9):
        acc = y[o][0]
        for k in range(1, G):
            acc = acc + y[o][k]
        out_ref[:, o, :] = acc


_TABW = 16


def _make_sc_gather(total):
    info = plsc.get_sparse_core_info()
    nw = info.num_cores * info.num_subcores
    per_w = total // nw
    mesh = plsc.VectorSubcoreMesh(core_axis_name="c", subcore_axis_name="s")

    @functools.partial(
        pl.kernel, mesh=mesh,
        out_type=jax.ShapeDtypeStruct((total, _TABW), jnp.float32),
        scratch_types=[
            pltpu.VMEM((per_w,), jnp.int32),
            pltpu.VMEM((per_w, _TABW), jnp.float32),
            pltpu.SemaphoreType.DMA,
        ],
        compiler_params=pltpu.CompilerParams(use_tc_tiling_on_sc=False),
    )
    def sc_gather(tab_hbm, idx_hbm, out_hbm, idx_v, rows_v, sem):
        wid = jax.lax.axis_index("s") * info.num_cores + jax.lax.axis_index("c")
        base = wid * per_w
        pltpu.sync_copy(idx_hbm.at[pl.ds(base, per_w)], idx_v)
        pltpu.async_copy(tab_hbm.at[idx_v], rows_v, sem).wait()
        pltpu.sync_copy(rows_v, out_hbm.at[pl.ds(base, per_w)])

    return sc_gather


def kernel(center, conv1_w, bn1_g, bn1_b, conv2_w, conv2_b, bn2_g, bn2_b,
           conv3_w, conv3_b):
    b, _, n = center.shape
    xyz = jnp.transpose(center, (0, 2, 1))

    knn = pl.pallas_call(
        functools.partial(_knn_kernel, n=n),
        grid=(b, n // QBLK),
        in_specs=[
            pl.BlockSpec((1, 3, QBLK), lambda bi, qi: (bi, 0, qi)),
            pl.BlockSpec((1, n, 3), lambda bi, qi: (bi, 0, 0)),
        ],
        out_specs=pl.BlockSpec((1, G, QBLK), lambda bi, qi: (bi, 0, qi)),
        out_shape=jax.ShapeDtypeStruct((b, G, n), jnp.int32),
    )
    idx = knn(center, xyz)

    total = b * G * n
    tab = jnp.zeros((b * n, _TABW), jnp.float32)
    tab = tab.at[:, :3].set(xyz.reshape(b * n, 3))
    rows = _make_sc_gather(total)(tab, idx.reshape(-1))
    gx = rows[:, 0].reshape(b, G, n)
    gy = rows[:, 1].reshape(b, G, n)
    gz = rows[:, 2].reshape(b, G, n)

    smem = pl.BlockSpec(memory_space=pltpu.SMEM)
    vmem = pl.BlockSpec(memory_space=pltpu.VMEM)
    out = pl.pallas_call(
        functools.partial(_umbrella_kernel, b=b, n=n),
        in_specs=[vmem, vmem, vmem, vmem,
                  smem, smem, smem, smem, smem, smem, smem, smem, smem],
        out_specs=vmem,
        out_shape=jax.ShapeDtypeStruct((b, 9, n), jnp.float32),
    )(gx, gy, gz, center,
      conv1_w, bn1_g, bn1_b,
      conv2_w, conv2_b, bn2_g, bn2_b,
      conv3_w, conv3_b)
    return out
